# baseline (speedup 1.0000x reference)
import functools

import jax
import jax.numpy as jnp
from jax import lax
from jax.experimental import pallas as pl
from jax.experimental.pallas import tpu as pltpu
from jax.experimental.pallas import tpu_sc as plsc

_D = 64
_DP = 128
_GRP = 128
_K = 5
_CHUNK = _GRP * _K
_NC = 2
_NS = 16
_NW = _NC * _NS
_BR = 2048


def _tw_block(x_ref, o_ref):
    o_ref[:, 0:_D] = x_ref[...].T


def _transpose_widen(table_t):
    v = table_t.shape[1]
    return pl.pallas_call(
        _tw_block,
        grid=(pl.cdiv(v, _BR),),
        in_specs=[pl.BlockSpec((_D, _BR), lambda i: (0, i))],
        out_specs=pl.BlockSpec((_BR, _DP), lambda i: (i, 0)),
        out_shape=jax.ShapeDtypeStruct((v, _DP), jnp.float32),
    )(table_t)


@jax.jit
def _sc_gather(table_wide, idx_groups):
    num_groups = idx_groups.shape[0]
    n = num_groups * _GRP
    rows_per_w = n // _NW
    nchunks = rows_per_w // _CHUNK
    mesh = plsc.VectorSubcoreMesh(core_axis_name="c", subcore_axis_name="s")

    @functools.partial(
        pl.kernel,
        out_type=jax.ShapeDtypeStruct((n, _D), jnp.float32),
        mesh=mesh,
        scratch_types=[
            pltpu.VMEM((_K, _GRP), jnp.int32),
            pltpu.VMEM((_CHUNK, _DP), jnp.float32),
            pltpu.SemaphoreType.DMA,
        ],
        compiler_params=pltpu.CompilerParams(use_tc_tiling_on_sc=False),
    )
    def k(table_hbm, idx_hbm, out_hbm, idx_v, rows_v, gsem):
        wid = lax.axis_index("s") * _NC + lax.axis_index("c")
        gbase = wid * (rows_per_w // _GRP)

        def body(c, carry):
            g0 = gbase + c * _K
            pltpu.sync_copy(idx_hbm.at[pl.ds(g0, _K)], idx_v)
            copies = [
                pltpu.async_copy(
                    table_hbm.at[idx_v.at[j]],
                    rows_v.at[pl.ds(j * _GRP, _GRP)],
                    gsem,
                )
                for j in range(_K)
            ]
            for cp in copies:
                cp.wait()
            pltpu.sync_copy(rows_v.at[:, pl.ds(0, _D)],
                            out_hbm.at[pl.ds(g0 * _GRP, _CHUNK)])
            return carry

        lax.fori_loop(0, nchunks, body, 0)

    return k(table_wide, idx_groups)


def kernel(input_tensor, indices):
    b, s = indices.shape
    n = b * s
    table_wide = jnp.pad(input_tensor, ((0, 0), (0, _DP - _D)))
    idx_groups = jnp.minimum(
        indices.astype(jnp.int32).reshape(n // _GRP, _GRP),
        input_tensor.shape[0] - 1,
    )
    out = _sc_gather(table_wide, idx_groups)
    return out.reshape(b, s, _D)


# speedup vs baseline: 1.0940x; 1.0940x over previous
import functools

import jax
import jax.numpy as jnp
from jax import lax
from jax.experimental import layout as jlayout
from jax.experimental import pallas as pl
from jax.experimental.pallas import tpu as pltpu
from jax.experimental.pallas import tpu_sc as plsc

_D = 64
_DP = 128
_GRP = 128
_K = 5
_CHUNK = _GRP * _K
_NC = 2
_NS = 16
_NW = _NC * _NS
_BR = 2048


def _tw_block(x_ref, o_ref):
    o_ref[:, 0:_D] = x_ref[...].T


def _transpose_widen(table_t):
    v = table_t.shape[1]
    return pl.pallas_call(
        _tw_block,
        grid=(pl.cdiv(v, _BR),),
        in_specs=[pl.BlockSpec((_D, _BR), lambda i: (0, i))],
        out_specs=pl.BlockSpec((_BR, _DP), lambda i: (i, 0)),
        out_shape=jax.ShapeDtypeStruct((v, _DP), jnp.float32),
    )(table_t)


@jax.jit
def _sc_gather(table_wide, idx_groups):
    num_groups = idx_groups.shape[0]
    n = num_groups * _GRP
    rows_per_w = n // _NW
    nchunks = rows_per_w // _CHUNK
    mesh = plsc.VectorSubcoreMesh(core_axis_name="c", subcore_axis_name="s")

    @functools.partial(
        pl.kernel,
        out_type=jax.ShapeDtypeStruct((n, _D), jnp.float32),
        mesh=mesh,
        scratch_types=[
            pltpu.VMEM((_K, _GRP), jnp.int32),
            pltpu.VMEM((_CHUNK, _DP), jnp.float32),
            pltpu.SemaphoreType.DMA,
        ],
        compiler_params=pltpu.CompilerParams(use_tc_tiling_on_sc=False),
    )
    def k(table_hbm, idx_hbm, out_hbm, idx_v, rows_v, gsem):
        wid = lax.axis_index("s") * _NC + lax.axis_index("c")
        gbase = wid * (rows_per_w // _GRP)

        def body(c, carry):
            g0 = gbase + c * _K
            pltpu.sync_copy(idx_hbm.at[pl.ds(g0, _K)], idx_v)
            copies = [
                pltpu.async_copy(
                    table_hbm.at[idx_v.at[j]],
                    rows_v.at[pl.ds(j * _GRP, _GRP)],
                    gsem,
                )
                for j in range(_K)
            ]
            for cp in copies:
                cp.wait()
            pltpu.sync_copy(rows_v.at[:, pl.ds(0, _D)],
                            out_hbm.at[pl.ds(g0 * _GRP, _CHUNK)])
            return carry

        lax.fori_loop(0, nchunks, body, 0)

    return k(table_wide, idx_groups)


def kernel(input_tensor, indices):
    b, s = indices.shape
    n = b * s
    table_wide = _transpose_widen(input_tensor.T)
    idx_groups = jnp.minimum(
        indices.astype(jnp.int32).reshape(n // _GRP, _GRP),
        input_tensor.shape[0] - 1,
    )
    out = _sc_gather(table_wide, idx_groups)
    out3 = out.reshape(b, s, _D)
    try:
        fmt = jlayout.Format(
            jlayout.Layout(major_to_minor=(0, 1, 2), tiling=((1024,),)),
            jax.sharding.SingleDeviceSharding(jax.devices()[0]),
        )
        out3 = jlayout.with_layout_constraint(out3, fmt)
    except Exception:
        pass
    return out3


# speedup vs baseline: 1.5305x; 1.3990x over previous
import functools

import jax
import jax.numpy as jnp
from jax import lax
from jax.experimental import pallas as pl
from jax.experimental.pallas import tpu as pltpu
from jax.experimental.pallas import tpu_sc as plsc

_D = 64
_DP = 128
_GRP = 128
_K = 5
_CHUNK = _GRP * _K
_NC = 2
_NS = 16
_NW = _NC * _NS
_BR = 8192


def _tw_block(x_ref, o_ref):
    o_ref[:, 0:_D] = x_ref[...].T


def _transpose_widen(table_t):
    v = table_t.shape[1]
    return pl.pallas_call(
        _tw_block,
        grid=(pl.cdiv(v, _BR),),
        in_specs=[pl.BlockSpec((_D, _BR), lambda i: (0, i))],
        out_specs=pl.BlockSpec((_BR, _DP), lambda i: (i, 0)),
        out_shape=jax.ShapeDtypeStruct((v, _DP), jnp.float32),
    )(table_t)


@jax.jit
def _sc_gather(table_wide, idx_groups):
    num_groups = idx_groups.shape[0]
    n = num_groups * _GRP
    rows_per_w = n // _NW
    nchunks = rows_per_w // _CHUNK
    mesh = plsc.VectorSubcoreMesh(core_axis_name="c", subcore_axis_name="s")

    @functools.partial(
        pl.kernel,
        out_type=jax.ShapeDtypeStruct((n, _D), jnp.float32),
        mesh=mesh,
        scratch_types=[
            pltpu.VMEM((_K, _GRP), jnp.int32),
            pltpu.VMEM((_CHUNK, _DP), jnp.float32),
            pltpu.SemaphoreType.DMA,
        ],
        compiler_params=pltpu.CompilerParams(use_tc_tiling_on_sc=False),
    )
    def k(table_hbm, idx_hbm, out_hbm, idx_v, rows_v, gsem):
        wid = lax.axis_index("s") * _NC + lax.axis_index("c")
        gbase = wid * (rows_per_w // _GRP)

        def body(c, carry):
            g0 = gbase + c * _K
            pltpu.sync_copy(idx_hbm.at[pl.ds(g0, _K)], idx_v)
            copies = [
                pltpu.async_copy(
                    table_hbm.at[idx_v.at[j]],
                    rows_v.at[pl.ds(j * _GRP, _GRP)],
                    gsem,
                )
                for j in range(_K)
            ]
            for cp in copies:
                cp.wait()
            pltpu.sync_copy(rows_v.at[:, pl.ds(0, _D)],
                            out_hbm.at[pl.ds(g0 * _GRP, _CHUNK)])
            return carry

        lax.fori_loop(0, nchunks, body, 0)

    return k(table_wide, idx_groups)


def kernel(input_tensor, indices):
    b, s = indices.shape
    n = b * s
    table_wide = _transpose_widen(input_tensor.T)
    idx_groups = jnp.minimum(
        indices.astype(jnp.int32).reshape(n // _GRP, _GRP),
        input_tensor.shape[0] - 1,
    )
    out = _sc_gather(table_wide, idx_groups)
    return out.reshape(b, s, _D)


# speedup vs baseline: 1.6194x; 1.0581x over previous
import functools

import jax
import jax.numpy as jnp
from jax import lax
from jax.experimental import pallas as pl
from jax.experimental.pallas import tpu as pltpu
from jax.experimental.pallas import tpu_sc as plsc

_D = 64
_DP = 128
_GRP = 128
_K = 5
_CHUNK = _GRP * _K
_NC = 2
_NS = 16
_NW = _NC * _NS
_BR = 32768


def _tw_block(x_ref, o_ref):
    o_ref[:, 0:_D] = x_ref[...].T


def _transpose_widen(table_t):
    v = table_t.shape[1]
    return pl.pallas_call(
        _tw_block,
        grid=(pl.cdiv(v, _BR),),
        in_specs=[pl.BlockSpec((_D, _BR), lambda i: (0, i))],
        out_specs=pl.BlockSpec((_BR, _DP), lambda i: (i, 0)),
        out_shape=jax.ShapeDtypeStruct((v, _DP), jnp.float32),
    )(table_t)


@jax.jit
def _sc_gather(table_wide, idx_groups):
    num_groups = idx_groups.shape[0]
    n = num_groups * _GRP
    rows_per_w = n // _NW
    nchunks = rows_per_w // _CHUNK
    mesh = plsc.VectorSubcoreMesh(core_axis_name="c", subcore_axis_name="s")

    @functools.partial(
        pl.kernel,
        out_type=jax.ShapeDtypeStruct((n, _D), jnp.float32),
        mesh=mesh,
        scratch_types=[
            pltpu.VMEM((_K, _GRP), jnp.int32),
            pltpu.VMEM((_CHUNK, _DP), jnp.float32),
            pltpu.SemaphoreType.DMA,
        ],
        compiler_params=pltpu.CompilerParams(use_tc_tiling_on_sc=False),
    )
    def k(table_hbm, idx_hbm, out_hbm, idx_v, rows_v, gsem):
        wid = lax.axis_index("s") * _NC + lax.axis_index("c")
        gbase = wid * (rows_per_w // _GRP)

        def body(c, carry):
            g0 = gbase + c * _K
            pltpu.sync_copy(idx_hbm.at[pl.ds(g0, _K)], idx_v)
            copies = [
                pltpu.async_copy(
                    table_hbm.at[idx_v.at[j]],
                    rows_v.at[pl.ds(j * _GRP, _GRP)],
                    gsem,
                )
                for j in range(_K)
            ]
            for cp in copies:
                cp.wait()
            pltpu.sync_copy(rows_v.at[:, pl.ds(0, _D)],
                            out_hbm.at[pl.ds(g0 * _GRP, _CHUNK)])
            return carry

        lax.fori_loop(0, nchunks, body, 0)

    return k(table_wide, idx_groups)


def kernel(input_tensor, indices):
    b, s = indices.shape
    n = b * s
    table_wide = _transpose_widen(input_tensor.T)
    idx_groups = jnp.minimum(
        indices.astype(jnp.int32).reshape(n // _GRP, _GRP),
        input_tensor.shape[0] - 1,
    )
    out = _sc_gather(table_wide, idx_groups)
    return out.reshape(b, s, _D)
